# baseline (speedup 1.0000x reference)
import jax
import jax.numpy as jnp
from jax.experimental import pallas as pl
from jax.experimental.pallas import tpu as pltpu

_S = 5
_H = 24
_TB = 32768


def _round_up(x, m):
    return ((x + m - 1) // m) * m


def _mlp_kernel(xT_ref, w1a_ref, w2a_ref, w3a_ref, o_ref):
    f32 = jnp.float32
    bf16 = jnp.bfloat16
    ones = jnp.ones((1, _TB), bf16)
    xaug = jnp.concatenate([xT_ref[...], ones], axis=0)
    h = jnp.dot(w1a_ref[...], xaug, preferred_element_type=f32)
    h = jnp.maximum(h, 0.0).astype(bf16)
    h = jnp.concatenate([h, ones], axis=0)
    h = jnp.dot(w2a_ref[...], h, preferred_element_type=f32)
    h = jnp.maximum(h, 0.0).astype(bf16)
    h = jnp.concatenate([h, ones], axis=0)
    h = jnp.dot(w3a_ref[...], h, preferred_element_type=f32)
    o_ref[...] = jnp.maximum(h, 0.0).astype(bf16)


def kernel(x, w1, b1, w2, b2, w3, b3):
    B = x.shape[0]
    B_pad = _round_up(B, 2 * _TB)
    xT = x.T.astype(jnp.bfloat16)
    if B_pad != B:
        xT = jnp.pad(xT, ((0, 0), (0, B_pad - B)))
    num_tiles = B_pad // _TB

    bf16 = jnp.bfloat16
    w1a = jnp.concatenate([w1, b1], axis=1).astype(bf16)
    w2a = jnp.concatenate([w2, b2], axis=1).astype(bf16)
    w3a = jnp.concatenate([w3, b3], axis=1).astype(bf16)

    resident = lambda shape: pl.BlockSpec(shape, lambda i: (0, 0))
    flops = 2 * B_pad * (_S * _H + _H * _H + _H * _S)
    bytes_accessed = B_pad * (_S + _S) * 2 + 2 * (
        _S * _H + _H * _H + _S * _H + 2 * _H + _S)

    oT = pl.pallas_call(
        _mlp_kernel,
        out_shape=jax.ShapeDtypeStruct((_S, B_pad), bf16),
        grid=(num_tiles,),
        in_specs=[
            pl.BlockSpec((_S, _TB), lambda i: (0, i)),
            resident((_H, _S + 1)),
            resident((_H, _H + 1)),
            resident((_S, _H + 1)),
        ],
        out_specs=pl.BlockSpec((_S, _TB), lambda i: (0, i)),
        compiler_params=pltpu.CompilerParams(
            dimension_semantics=("parallel",),
            vmem_limit_bytes=100 * 1024 * 1024,
        ),
        cost_estimate=pl.CostEstimate(
            flops=flops, transcendentals=0, bytes_accessed=bytes_accessed),
    )(xT, w1a, w2a, w3a)
    return oT[:, :B].T.astype(jnp.float32)


# speedup vs baseline: 1.3400x; 1.3400x over previous
import jax
import jax.numpy as jnp
from jax.experimental import pallas as pl
from jax.experimental.pallas import tpu as pltpu

_S = 5
_H = 24
_TB = 65536


def _round_up(x, m):
    return ((x + m - 1) // m) * m


def _mlp_kernel(xT_ref, w1a_ref, w2a_ref, w3a_ref, o_ref):
    f32 = jnp.float32
    ones = jnp.ones((1, _TB), f32)
    h = jnp.dot(w1a_ref[...], xT_ref[...], preferred_element_type=f32)
    h = jnp.maximum(h, 0.0)
    h = jnp.concatenate([h, ones], axis=0)
    h = jnp.dot(w2a_ref[...], h, preferred_element_type=f32)
    h = jnp.maximum(h, 0.0)
    h = jnp.concatenate([h, ones], axis=0)
    h = jnp.dot(w3a_ref[...], h, preferred_element_type=f32)
    o_ref[...] = jnp.maximum(h, 0.0)


def kernel(x, w1, b1, w2, b2, w3, b3):
    B = x.shape[0]
    B_pad = _round_up(B, 2 * _TB)
    xT = jnp.concatenate([x.T, jnp.ones((1, B), jnp.float32)], axis=0)
    if B_pad != B:
        xT = jnp.pad(xT, ((0, 0), (0, B_pad - B), ))
    num_tiles = B_pad // _TB

    w1a = jnp.concatenate([w1, b1], axis=1)
    w2a = jnp.concatenate([w2, b2], axis=1)
    w3a = jnp.concatenate([w3, b3], axis=1)

    resident = lambda shape: pl.BlockSpec(shape, lambda i: (0, 0))
    flops = 2 * B_pad * (_S * _H + _H * _H + _H * _S)
    bytes_accessed = B_pad * (_S + _S) * 4 + 4 * (
        _S * _H + _H * _H + _S * _H + 2 * _H + _S)

    oT = pl.pallas_call(
        _mlp_kernel,
        out_shape=jax.ShapeDtypeStruct((_S, B_pad), jnp.float32),
        grid=(num_tiles,),
        in_specs=[
            pl.BlockSpec((_S + 1, _TB), lambda i: (0, i)),
            resident((_H, _S + 1)),
            resident((_H, _H + 1)),
            resident((_S, _H + 1)),
        ],
        out_specs=pl.BlockSpec((_S, _TB), lambda i: (0, i)),
        compiler_params=pltpu.CompilerParams(
            dimension_semantics=("parallel",),
            vmem_limit_bytes=100 * 1024 * 1024,
        ),
        cost_estimate=pl.CostEstimate(
            flops=flops, transcendentals=0, bytes_accessed=bytes_accessed),
    )(xT, w1a, w2a, w3a)
    return oT[:, :B].T


# speedup vs baseline: 1.9768x; 1.4752x over previous
import jax
import jax.numpy as jnp
from jax.experimental import pallas as pl
from jax.experimental.pallas import tpu as pltpu

_S = 5
_H = 24
_TB = 65536


def _round_up(x, m):
    return ((x + m - 1) // m) * m


def _mlp_kernel(xT_ref, w1a_ref, w2a_ref, w3a_ref, o_ref):
    f32 = jnp.float32
    ones = jnp.ones((1, _TB), f32)
    xaug = jnp.concatenate([xT_ref[...], ones], axis=0)
    h = jnp.dot(w1a_ref[...], xaug, preferred_element_type=f32)
    h = jnp.maximum(h, 0.0)
    h = jnp.concatenate([h, ones], axis=0)
    h = jnp.dot(w2a_ref[...], h, preferred_element_type=f32)
    h = jnp.maximum(h, 0.0)
    h = jnp.concatenate([h, ones], axis=0)
    h = jnp.dot(w3a_ref[...], h, preferred_element_type=f32)
    o_ref[...] = jnp.maximum(h, 0.0)


def kernel(x, w1, b1, w2, b2, w3, b3):
    B = x.shape[0]
    B_pad = _round_up(B, 2 * _TB)
    xT = x.T
    if B_pad != B:
        xT = jnp.pad(xT, ((0, 0), (0, B_pad - B)))
    num_tiles = B_pad // _TB

    w1a = jnp.concatenate([w1, b1], axis=1)
    w2a = jnp.concatenate([w2, b2], axis=1)
    w3a = jnp.concatenate([w3, b3], axis=1)

    resident = lambda shape: pl.BlockSpec(shape, lambda i: (0, 0))
    flops = 2 * B_pad * (_S * _H + _H * _H + _H * _S)
    bytes_accessed = B_pad * (_S + _S) * 4 + 4 * (
        _S * _H + _H * _H + _S * _H + 2 * _H + _S)

    oT = pl.pallas_call(
        _mlp_kernel,
        out_shape=jax.ShapeDtypeStruct((_S, B_pad), jnp.float32),
        grid=(num_tiles,),
        in_specs=[
            pl.BlockSpec((_S, _TB), lambda i: (0, i)),
            resident((_H, _S + 1)),
            resident((_H, _H + 1)),
            resident((_S, _H + 1)),
        ],
        out_specs=pl.BlockSpec((_S, _TB), lambda i: (0, i)),
        compiler_params=pltpu.CompilerParams(
            dimension_semantics=("parallel",),
            vmem_limit_bytes=100 * 1024 * 1024,
        ),
        cost_estimate=pl.CostEstimate(
            flops=flops, transcendentals=0, bytes_accessed=bytes_accessed),
    )(xT, w1a, w2a, w3a)
    return oT[:, :B].T


# speedup vs baseline: 2.0829x; 1.0537x over previous
import jax
import jax.numpy as jnp
from jax.experimental import pallas as pl
from jax.experimental.pallas import tpu as pltpu

_S = 5
_H = 24
_TB = 65536


def _round_up(x, m):
    return ((x + m - 1) // m) * m


def _mlp_kernel(xT_ref, w1a_ref, w2a_ref, b2_ref, w3a_ref, b3_ref, o_ref):
    f32 = jnp.float32
    ones = jnp.ones((1, _TB), f32)
    xaug = jnp.concatenate([xT_ref[...], ones], axis=0)
    h = jnp.dot(w1a_ref[...], xaug, preferred_element_type=f32)
    h = jnp.maximum(h, 0.0)
    h = jnp.dot(w2a_ref[...], h, preferred_element_type=f32)
    h = jnp.maximum(h + b2_ref[...], 0.0)
    h = jnp.dot(w3a_ref[...], h, preferred_element_type=f32)
    o_ref[...] = jnp.maximum(h + b3_ref[...], 0.0)


def kernel(x, w1, b1, w2, b2, w3, b3):
    B = x.shape[0]
    B_pad = _round_up(B, 2 * _TB)
    xT = x.T
    if B_pad != B:
        xT = jnp.pad(xT, ((0, 0), (0, B_pad - B)))
    num_tiles = B_pad // _TB

    w1a = jnp.concatenate([w1, b1], axis=1)

    resident = lambda shape: pl.BlockSpec(shape, lambda i: (0, 0))
    flops = 2 * B_pad * (_S * _H + _H * _H + _H * _S)
    bytes_accessed = B_pad * (_S + _S) * 4 + 4 * (
        _S * _H + _H * _H + _S * _H + 2 * _H + _S)

    oT = pl.pallas_call(
        _mlp_kernel,
        out_shape=jax.ShapeDtypeStruct((_S, B_pad), jnp.float32),
        grid=(num_tiles,),
        in_specs=[
            pl.BlockSpec((_S, _TB), lambda i: (0, i)),
            resident((_H, _S + 1)),
            resident((_H, _H)),
            resident((_H, 1)),
            resident((_S, _H)),
            resident((_S, 1)),
        ],
        out_specs=pl.BlockSpec((_S, _TB), lambda i: (0, i)),
        compiler_params=pltpu.CompilerParams(
            dimension_semantics=("parallel",),
            vmem_limit_bytes=100 * 1024 * 1024,
        ),
        cost_estimate=pl.CostEstimate(
            flops=flops, transcendentals=0, bytes_accessed=bytes_accessed),
    )(xT, w1a, w2, b2, w3, b3)
    return oT[:, :B].T


# speedup vs baseline: 2.1328x; 1.0240x over previous
import jax
import jax.numpy as jnp
from jax.experimental import pallas as pl
from jax.experimental.pallas import tpu as pltpu

_S = 5
_H = 24
_TB = 131072


def _round_up(x, m):
    return ((x + m - 1) // m) * m


def _mlp_kernel(xT_ref, w1a_ref, w2a_ref, b2_ref, w3a_ref, b3_ref, o_ref):
    f32 = jnp.float32
    ones = jnp.ones((1, _TB), f32)
    xaug = jnp.concatenate([xT_ref[...], ones], axis=0)
    h = jnp.dot(w1a_ref[...], xaug, preferred_element_type=f32)
    h = jnp.maximum(h, 0.0)
    h = jnp.dot(w2a_ref[...], h, preferred_element_type=f32)
    h = jnp.maximum(h + b2_ref[...], 0.0)
    h = jnp.dot(w3a_ref[...], h, preferred_element_type=f32)
    o_ref[...] = jnp.maximum(h + b3_ref[...], 0.0)


def kernel(x, w1, b1, w2, b2, w3, b3):
    B = x.shape[0]
    B_pad = _round_up(B, 2 * _TB)
    xT = x.T
    if B_pad != B:
        xT = jnp.pad(xT, ((0, 0), (0, B_pad - B)))
    num_tiles = B_pad // _TB

    w1a = jnp.concatenate([w1, b1], axis=1)

    resident = lambda shape: pl.BlockSpec(shape, lambda i: (0, 0))
    flops = 2 * B_pad * (_S * _H + _H * _H + _H * _S)
    bytes_accessed = B_pad * (_S + _S) * 4 + 4 * (
        _S * _H + _H * _H + _S * _H + 2 * _H + _S)

    oT = pl.pallas_call(
        _mlp_kernel,
        out_shape=jax.ShapeDtypeStruct((_S, B_pad), jnp.float32),
        grid=(num_tiles,),
        in_specs=[
            pl.BlockSpec((_S, _TB), lambda i: (0, i)),
            resident((_H, _S + 1)),
            resident((_H, _H)),
            resident((_H, 1)),
            resident((_S, _H)),
            resident((_S, 1)),
        ],
        out_specs=pl.BlockSpec((_S, _TB), lambda i: (0, i)),
        compiler_params=pltpu.CompilerParams(
            dimension_semantics=("parallel",),
            vmem_limit_bytes=100 * 1024 * 1024,
        ),
        cost_estimate=pl.CostEstimate(
            flops=flops, transcendentals=0, bytes_accessed=bytes_accessed),
    )(xT, w1a, w2, b2, w3, b3)
    return oT[:, :B].T
